# Initial kernel scaffold; baseline (speedup 1.0000x reference)
import functools

import jax
import jax.numpy as jnp
from jax import lax
from jax.experimental import pallas as pl
from jax.experimental.pallas import tpu as pltpu
from jax.experimental.pallas import tpu_sc as plsc

N_NODES = 100000
D = 32
E = 1280000

NC, NS = 2, 16
W = NC * NS
EW = E // W
STREAM_E = 64
IDXW = 2 * STREAM_E
SPG = 5
GE = STREAM_E * SPG
NG = EW // GE
IDX_ROWS_W = EW * 2 // IDXW


def _normalize(emb):
    def body(x_ref, o_ref):
        x = x_ref[...]
        n = jnp.sqrt(jnp.sum(x * x, axis=1, keepdims=True))
        o_ref[...] = x / jnp.maximum(n, 1e-12)

    return pl.pallas_call(
        body,
        grid=(50,),
        in_specs=[pl.BlockSpec((N_NODES // 50, D), lambda i: (i, 0))],
        out_specs=pl.BlockSpec((N_NODES // 50, D), lambda i: (i, 0)),
        out_shape=jax.ShapeDtypeStruct((N_NODES, D), jnp.float32),
    )(emb)


def _sc_body(table, idx2d, out, idx_v, rows_v, out_v, sem):
    wid = lax.axis_index("s") * NC + lax.axis_index("c")
    idx_base = wid * IDX_ROWS_W
    out_base = wid * EW

    def issue(g, buf):
        pltpu.sync_copy(idx2d.at[pl.ds(idx_base + g * SPG, SPG)], idx_v.at[buf])
        for j in range(SPG):
            pltpu.async_copy(table.at[idx_v.at[buf, j]], rows_v.at[buf, j],
                             sem.at[buf])

    issue(0, 0)

    def group_body(g, carry):
        cur = lax.rem(g, 2)
        nxt = 1 - cur

        @pl.when(g < NG - 1)
        def _():
            issue(g + 1, nxt)

        for j in range(SPG):
            pltpu.make_async_copy(table.at[pl.ds(0, IDXW)], rows_v.at[cur, j],
                                  sem.at[cur]).wait()

        def stream_body(j, c):
            r = rows_v.at[cur, j]
            for e in range(STREAM_E):
                s0 = r[2 * e, 0:16]
                s1 = r[2 * e, 16:32]
                d0 = r[2 * e + 1, 0:16]
                d1 = r[2 * e + 1, 16:32]
                p = s0 * d0 + s1 * d1
                out_v[cur, j * STREAM_E + e] = jnp.sum(p)
            return c

        lax.fori_loop(0, SPG, stream_body, 0)
        pltpu.sync_copy(out_v.at[cur], out.at[pl.ds(out_base + g * GE, GE)])
        return carry

    lax.fori_loop(0, NG, group_body, 0)


def kernel(embeddings, pos_edges, neg_edges):
    emb_n = _normalize(embeddings)
    edges = jnp.concatenate([pos_edges.reshape(-1), neg_edges.reshape(-1)])
    idx2d = edges.reshape(E * 2 // IDXW, IDXW)
    sc = pl.kernel(
        _sc_body,
        out_type=jax.ShapeDtypeStruct((E,), jnp.float32),
        mesh=plsc.VectorSubcoreMesh(core_axis_name="c", subcore_axis_name="s"),
        scratch_types=[
            pltpu.VMEM((2, SPG, IDXW), jnp.int32),
            pltpu.VMEM((2, SPG, IDXW, D), jnp.float32),
            pltpu.VMEM((2, GE), jnp.float32),
            pltpu.SemaphoreType.DMA((2,)),
        ],
    )
    return sc(emb_n, idx2d)


# speedup vs baseline: 4.1141x; 4.1141x over previous
import functools

import jax
import jax.numpy as jnp
from jax import lax
from jax.experimental import pallas as pl
from jax.experimental.pallas import tpu as pltpu
from jax.experimental.pallas import tpu_sc as plsc

N_NODES = 100000
D = 32
E = 1280000

NC, NS = 2, 16
W = NC * NS
EW = E // W
STREAM_E = 64
IDXW = 2 * STREAM_E
SPG = 5
GE = STREAM_E * SPG
NG = EW // GE
IDX_ROWS_W = EW * 2 // IDXW


def _normalize(emb):
    def body(x_ref, o_ref):
        x = x_ref[...]
        n = jnp.sqrt(jnp.sum(x * x, axis=1, keepdims=True))
        o_ref[...] = x / jnp.maximum(n, 1e-12)

    return pl.pallas_call(
        body,
        grid=(50,),
        in_specs=[pl.BlockSpec((N_NODES // 50, D), lambda i: (i, 0))],
        out_specs=pl.BlockSpec((N_NODES // 50, D), lambda i: (i, 0)),
        out_shape=jax.ShapeDtypeStruct((N_NODES, D), jnp.float32),
    )(emb)


def _sc_body(table, eidx, out, idx_v, rows_v, out_v, sem):
    wid = lax.axis_index("s") * NC + lax.axis_index("c")
    idx_base = wid * EW * 2
    out_base = wid * EW

    def issue(g, buf):
        pltpu.sync_copy(eidx.at[pl.ds(idx_base + g * (2 * GE), 2 * GE)],
                        idx_v.at[buf])
        for j in range(SPG):
            pltpu.async_copy(table.at[idx_v.at[buf, pl.ds(j * IDXW, IDXW)]],
                             rows_v.at[buf, j], sem.at[buf])

    issue(0, 0)

    def group_body(g, carry):
        cur = lax.rem(g, 2)
        nxt = 1 - cur

        @pl.when(g < NG - 1)
        def _():
            issue(g + 1, nxt)

        for j in range(SPG):
            pltpu.make_async_copy(table.at[pl.ds(0, IDXW)], rows_v.at[cur, j],
                                  sem.at[cur]).wait()

        row2 = 2 * lax.iota(jnp.int32, 16)

        def stream_body(j, c):
            r = rows_v.at[cur, j]
            for b in range(STREAM_E // 16):
                rs = row2 + (32 * b)
                rd = rs + 1
                acc = jnp.zeros((16,), jnp.float32)
                for d in range(D):
                    cd = jnp.full((16,), d, jnp.int32)
                    sv = plsc.load_gather(r, [rs, cd])
                    dv = plsc.load_gather(r, [rd, cd])
                    acc = acc + sv * dv
                out_v[cur, pl.ds(j * STREAM_E + b * 16, 16)] = acc
            return c

        lax.fori_loop(0, SPG, stream_body, 0)
        pltpu.sync_copy(out_v.at[cur], out.at[pl.ds(out_base + g * GE, GE)])
        return carry

    lax.fori_loop(0, NG, group_body, 0)


def kernel(embeddings, pos_edges, neg_edges):
    emb_n = _normalize(embeddings)
    edges = jnp.concatenate([pos_edges.reshape(-1), neg_edges.reshape(-1)])
    sc = pl.kernel(
        _sc_body,
        out_type=jax.ShapeDtypeStruct((E,), jnp.float32),
        mesh=plsc.VectorSubcoreMesh(core_axis_name="c", subcore_axis_name="s"),
        scratch_types=[
            pltpu.VMEM((2, 2 * GE), jnp.int32),
            pltpu.VMEM((2, SPG, IDXW, D), jnp.float32),
            pltpu.VMEM((2, GE), jnp.float32),
            pltpu.SemaphoreType.DMA((2,)),
        ],
        compiler_params=pltpu.CompilerParams(
            needs_layout_passes=False, use_tc_tiling_on_sc=False),
    )
    return sc(emb_n, edges)


# speedup vs baseline: 23.0732x; 5.6083x over previous
import functools

import jax
import jax.numpy as jnp
from jax import lax
from jax.experimental import pallas as pl
from jax.experimental.pallas import tpu as pltpu
from jax.experimental.pallas import tpu_sc as plsc

N_NODES = 100000
D = 32
E = 1280000

NC, NS = 2, 16
W = NC * NS
EW = E // W
GE = 320
NG = EW // GE
SPG = 4
SR = GE // SPG


def _normalize(emb):
    def body(x_ref, o_ref):
        x = x_ref[...]
        n = jnp.sqrt(jnp.sum(x * x, axis=1, keepdims=True))
        o_ref[...] = x / jnp.maximum(n, 1e-12)

    return pl.pallas_call(
        body,
        grid=(50,),
        in_specs=[pl.BlockSpec((N_NODES // 50, D), lambda i: (i, 0))],
        out_specs=pl.BlockSpec((N_NODES // 50, D), lambda i: (i, 0)),
        out_shape=jax.ShapeDtypeStruct((N_NODES, D), jnp.float32),
    )(emb)


def _sc_body(table, sidx, didx, out, idx_v, src_v, dst_v, out_v, sem):
    wid = lax.axis_index("s") * NC + lax.axis_index("c")
    ebase = wid * EW

    def issue(g, buf):
        base = ebase + g * GE
        pltpu.sync_copy(sidx.at[pl.ds(base, GE)], idx_v.at[buf, 0])
        pltpu.sync_copy(didx.at[pl.ds(base, GE)], idx_v.at[buf, 1])
        for j in range(SPG):
            pltpu.async_copy(table.at[idx_v.at[buf, 0, pl.ds(j * SR, SR)]],
                             src_v.at[buf, pl.ds(j * SR, SR)], sem.at[buf])
            pltpu.async_copy(table.at[idx_v.at[buf, 1, pl.ds(j * SR, SR)]],
                             dst_v.at[buf, pl.ds(j * SR, SR)], sem.at[buf])

    issue(0, 0)
    lanes = lax.iota(jnp.int32, 16)

    def group_body(g, carry):
        cur = lax.rem(g, 2)
        nxt = 1 - cur

        @pl.when(g < NG - 1)
        def _():
            issue(g + 1, nxt)

        for j in range(2 * SPG):
            pltpu.make_async_copy(table.at[pl.ds(0, SR)],
                                  src_v.at[cur, pl.ds(0, SR)],
                                  sem.at[cur]).wait()

        sref = src_v.at[cur]
        dref = dst_v.at[cur]

        def batch_body(b, c):
            row = lanes + 16 * b
            acc = jnp.zeros((16,), jnp.float32)
            for d in range(D):
                col = (lanes + d) & 31
                sv = plsc.load_gather(sref, [row, col])
                dv = plsc.load_gather(dref, [row, col])
                acc = acc + sv * dv
            out_v[cur, pl.ds(16 * b, 16)] = acc
            return c

        lax.fori_loop(0, GE // 16, batch_body, 0)
        pltpu.sync_copy(out_v.at[cur], out.at[pl.ds(ebase + g * GE, GE)])
        return carry

    lax.fori_loop(0, NG, group_body, 0)


def kernel(embeddings, pos_edges, neg_edges):
    emb_n = _normalize(embeddings)
    sidx = jnp.concatenate([pos_edges[:, 0], neg_edges[:, 0]])
    didx = jnp.concatenate([pos_edges[:, 1], neg_edges[:, 1]])
    sc = pl.kernel(
        _sc_body,
        out_type=jax.ShapeDtypeStruct((E,), jnp.float32),
        mesh=plsc.VectorSubcoreMesh(core_axis_name="c", subcore_axis_name="s"),
        scratch_types=[
            pltpu.VMEM((2, 2, GE), jnp.int32),
            pltpu.VMEM((2, GE, D), jnp.float32),
            pltpu.VMEM((2, GE, D), jnp.float32),
            pltpu.VMEM((2, GE), jnp.float32),
            pltpu.SemaphoreType.DMA((2,)),
        ],
        compiler_params=pltpu.CompilerParams(
            needs_layout_passes=False, use_tc_tiling_on_sc=False),
    )
    return sc(emb_n, sidx, didx)


# speedup vs baseline: 29.9648x; 1.2987x over previous
import functools

import jax
import jax.numpy as jnp
from jax import lax
from jax.experimental import pallas as pl
from jax.experimental.pallas import tpu as pltpu
from jax.experimental.pallas import tpu_sc as plsc

N_NODES = 100000
D = 32
E = 1280000

NC, NS = 2, 16
W = NC * NS
EW = E // W
GE = 320
NG = EW // GE
SPG = 4
SR = GE // SPG


def _normalize(emb):
    def body(x_ref, o_ref):
        x = x_ref[...]
        n = jnp.sqrt(jnp.sum(x * x, axis=1, keepdims=True))
        o_ref[...] = x / jnp.maximum(n, 1e-12)

    return pl.pallas_call(
        body,
        grid=(50,),
        in_specs=[pl.BlockSpec((N_NODES // 50, D), lambda i: (i, 0))],
        out_specs=pl.BlockSpec((N_NODES // 50, D), lambda i: (i, 0)),
        out_shape=jax.ShapeDtypeStruct((N_NODES, D), jnp.float32),
    )(emb)


def _sc_body(table, sidx, didx, out, idx_v, src_v, dst_v, out_v, sem, isem):
    wid = lax.axis_index("s") * NC + lax.axis_index("c")
    ebase = wid * EW

    def idx_copy(g, slot):
        base = ebase + g * GE
        pltpu.async_copy(sidx.at[pl.ds(base, GE)], idx_v.at[slot, 0],
                         isem.at[slot])
        pltpu.async_copy(didx.at[pl.ds(base, GE)], idx_v.at[slot, 1],
                         isem.at[slot])

    def idx_wait(slot):
        pltpu.make_async_copy(sidx.at[pl.ds(0, GE)], idx_v.at[slot, 0],
                              isem.at[slot]).wait()
        pltpu.make_async_copy(didx.at[pl.ds(0, GE)], idx_v.at[slot, 1],
                              isem.at[slot]).wait()

    def fire(g, slot, buf):
        for j in range(SPG):
            pltpu.async_copy(table.at[idx_v.at[slot, 0, pl.ds(j * SR, SR)]],
                             src_v.at[buf, pl.ds(j * SR, SR)], sem.at[buf])
            pltpu.async_copy(table.at[idx_v.at[slot, 1, pl.ds(j * SR, SR)]],
                             dst_v.at[buf, pl.ds(j * SR, SR)], sem.at[buf])

    idx_copy(0, 0)
    idx_copy(1, 1)
    idx_wait(0)
    fire(0, 0, 0)
    lanes = lax.iota(jnp.int32, 16)

    def group_body(g, carry):
        cur = lax.rem(g, 2)
        nxt = 1 - cur

        @pl.when(g < NG - 1)
        def _():
            idx_wait(lax.rem(g + 1, 3))
            fire(g + 1, lax.rem(g + 1, 3), nxt)

        @pl.when(g < NG - 2)
        def _():
            idx_copy(g + 2, lax.rem(g + 2, 3))

        for j in range(2 * SPG):
            pltpu.make_async_copy(table.at[pl.ds(0, SR)],
                                  src_v.at[cur, pl.ds(0, SR)],
                                  sem.at[cur]).wait()

        sref = src_v.at[cur]
        dref = dst_v.at[cur]

        def batch_body(b, c):
            row = lanes + 16 * b
            acc = jnp.zeros((16,), jnp.float32)
            for d in range(D):
                col = (lanes + d) & 31
                sv = plsc.load_gather(sref, [row, col])
                dv = plsc.load_gather(dref, [row, col])
                acc = acc + sv * dv
            out_v[cur, pl.ds(16 * b, 16)] = acc
            return c

        lax.fori_loop(0, GE // 16, batch_body, 0)
        pltpu.sync_copy(out_v.at[cur], out.at[pl.ds(ebase + g * GE, GE)])
        return carry

    lax.fori_loop(0, NG, group_body, 0)


def kernel(embeddings, pos_edges, neg_edges):
    emb_n = _normalize(embeddings)
    sidx = jnp.concatenate([pos_edges[:, 0], neg_edges[:, 0]])
    didx = jnp.concatenate([pos_edges[:, 1], neg_edges[:, 1]])
    sc = pl.kernel(
        _sc_body,
        out_type=jax.ShapeDtypeStruct((E,), jnp.float32),
        mesh=plsc.VectorSubcoreMesh(core_axis_name="c", subcore_axis_name="s"),
        scratch_types=[
            pltpu.VMEM((3, 2, GE), jnp.int32),
            pltpu.VMEM((2, GE, D), jnp.float32),
            pltpu.VMEM((2, GE, D), jnp.float32),
            pltpu.VMEM((2, GE), jnp.float32),
            pltpu.SemaphoreType.DMA((2,)),
            pltpu.SemaphoreType.DMA((3,)),
        ],
        compiler_params=pltpu.CompilerParams(
            needs_layout_passes=False, use_tc_tiling_on_sc=False),
    )
    return sc(emb_n, sidx, didx)
